# baseline (speedup 1.0000x reference)
import math

import jax
import jax.numpy as jnp
from jax.experimental import pallas as pl
from jax.experimental.pallas import tpu as pltpu

BQ = 256
HP = 2


def _attn_block_kernel(q_ref, k_ref, v_ref, m_ref, o_ref, kb_ref, vb_ref):
    dh = q_ref.shape[-1] // HP
    c = math.log2(math.e) / math.sqrt(dh)

    @pl.when(pl.program_id(1) == 0)
    def _():
        kb_ref[...] = k_ref[...].astype(jnp.bfloat16)
        vb_ref[...] = v_ref[...].astype(jnp.bfloat16)

    qp = (q_ref[...] * c).astype(jnp.bfloat16)
    kp = kb_ref[...]
    vp = vb_ref[...]
    outs = []
    for j in range(HP):
        qj = qp[:, j * dh:(j + 1) * dh]
        kj = kp[:, j * dh:(j + 1) * dh]
        vj = vp[:, j * dh:(j + 1) * dh]
        mj = m_ref[j]
        s2 = jax.lax.dot_general(
            qj, kj, (((1,), (1,)), ((), ())), preferred_element_type=jnp.float32
        )
        mx = jnp.max(s2, axis=-1, keepdims=True)
        p = jnp.exp2(s2 - mx) * mj
        d = jnp.sum(p, axis=-1, keepdims=True)
        o = jax.lax.dot_general(
            p.astype(jnp.bfloat16), vj, (((1,), (0,)), ((), ())),
            preferred_element_type=jnp.float32,
        )
        outs.append(o / d)
    o_ref[...] = jnp.concatenate(outs, axis=-1)


def kernel(query, key, value, mask):
    b, s, h, dh = query.shape
    hd = h * dh
    nq = s // BQ
    nh = h // HP

    qb = query.reshape(s, hd)
    kb = key.reshape(s, hd)
    vb = value.reshape(s, hd)

    out = pl.pallas_call(
        _attn_block_kernel,
        grid=(nh, nq),
        in_specs=[
            pl.BlockSpec((BQ, HP * dh), lambda hh, i: (i, hh)),
            pl.BlockSpec((s, HP * dh), lambda hh, i: (0, hh)),
            pl.BlockSpec((s, HP * dh), lambda hh, i: (0, hh)),
            pl.BlockSpec((HP, BQ, s), lambda hh, i: (hh, i, 0)),
        ],
        out_specs=pl.BlockSpec((BQ, HP * dh), lambda hh, i: (i, hh)),
        out_shape=jax.ShapeDtypeStruct((s, hd), jnp.float32),
        scratch_shapes=[
            pltpu.VMEM((s, HP * dh), jnp.bfloat16),
            pltpu.VMEM((s, HP * dh), jnp.bfloat16),
        ],
    )(qb, kb, vb, mask)

    return out.reshape(s, b, hd)


# speedup vs baseline: 1.1620x; 1.1620x over previous
import math

import jax
import jax.numpy as jnp
from jax.experimental import pallas as pl
from jax.experimental.pallas import tpu as pltpu

BQ = 256
HP = 2


def _attn_block_kernel(q_ref, k_ref, v_ref, m_ref, o_ref, kb_ref, vb_ref):
    dh = q_ref.shape[-1] // HP
    c = math.log2(math.e) / math.sqrt(dh)

    @pl.when(pl.program_id(1) == 0)
    def _():
        kb_ref[...] = k_ref[...].astype(jnp.bfloat16)
        vb_ref[...] = v_ref[...].astype(jnp.bfloat16)

    qp = (q_ref[...] * c).astype(jnp.bfloat16)
    kp = kb_ref[...]
    vp = vb_ref[...]
    outs = []
    for j in range(HP):
        qj = qp[:, j * dh:(j + 1) * dh]
        kj = kp[:, j * dh:(j + 1) * dh]
        vj = vp[:, j * dh:(j + 1) * dh]
        mj = m_ref[j]
        s2 = jax.lax.dot_general(
            qj, kj, (((1,), (1,)), ((), ())), preferred_element_type=jnp.float32
        )
        mx = jnp.max(s2, axis=-1, keepdims=True)
        p = jnp.exp2(s2 - mx) * mj
        d = jnp.sum(p, axis=-1, keepdims=True)
        o = jax.lax.dot_general(
            p.astype(jnp.bfloat16), vj, (((1,), (0,)), ((), ())),
            preferred_element_type=jnp.float32,
        )
        outs.append(o / d)
    o_ref[:, 0, :] = jnp.concatenate(outs, axis=-1)


def kernel(query, key, value, mask):
    b, s, h, dh = query.shape
    hd = h * dh
    nq = s // BQ
    nh = h // HP

    qb = query.reshape(s, hd)
    kb = key.reshape(s, hd)
    vb = value.reshape(s, hd)

    out = pl.pallas_call(
        _attn_block_kernel,
        grid=(nh, nq),
        in_specs=[
            pl.BlockSpec((BQ, HP * dh), lambda hh, i: (i, hh)),
            pl.BlockSpec((s, HP * dh), lambda hh, i: (0, hh)),
            pl.BlockSpec((s, HP * dh), lambda hh, i: (0, hh)),
            pl.BlockSpec((HP, BQ, s), lambda hh, i: (hh, i, 0)),
        ],
        out_specs=pl.BlockSpec((BQ, 1, HP * dh), lambda hh, i: (i, 0, hh)),
        out_shape=jax.ShapeDtypeStruct((s, b, hd), jnp.float32),
        scratch_shapes=[
            pltpu.VMEM((s, HP * dh), jnp.bfloat16),
            pltpu.VMEM((s, HP * dh), jnp.bfloat16),
        ],
    )(qb, kb, vb, mask)

    return out


# speedup vs baseline: 1.2421x; 1.0689x over previous
import math

import jax
import jax.numpy as jnp
from jax.experimental import pallas as pl
from jax.experimental.pallas import tpu as pltpu

BQ = 512
HP = 2


def _attn_block_kernel(q_ref, k_ref, v_ref, m_ref, o_ref, kb_ref, vb_ref):
    dh = q_ref.shape[-1] // HP
    c = math.log2(math.e) / math.sqrt(dh)

    @pl.when(pl.program_id(1) == 0)
    def _():
        kb_ref[...] = k_ref[...].astype(jnp.bfloat16)
        vb_ref[...] = v_ref[...].astype(jnp.bfloat16)

    qp = (q_ref[...] * c).astype(jnp.bfloat16)
    kp = kb_ref[...]
    vp = vb_ref[...]
    outs = []
    for j in range(HP):
        qj = qp[:, j * dh:(j + 1) * dh]
        kj = kp[:, j * dh:(j + 1) * dh]
        vj = vp[:, j * dh:(j + 1) * dh]
        mj = m_ref[j]
        s2 = jax.lax.dot_general(
            qj, kj, (((1,), (1,)), ((), ())), preferred_element_type=jnp.float32
        )
        mx = jnp.max(s2, axis=-1, keepdims=True)
        p = jnp.exp2(s2 - mx) * mj
        d = jnp.sum(p, axis=-1, keepdims=True)
        o = jax.lax.dot_general(
            p.astype(jnp.bfloat16), vj, (((1,), (0,)), ((), ())),
            preferred_element_type=jnp.float32,
        )
        outs.append(o / d)
    o_ref[:, 0, :] = jnp.concatenate(outs, axis=-1)


def kernel(query, key, value, mask):
    b, s, h, dh = query.shape
    hd = h * dh
    nq = s // BQ
    nh = h // HP

    qb = query.reshape(s, hd)
    kb = key.reshape(s, hd)
    vb = value.reshape(s, hd)

    out = pl.pallas_call(
        _attn_block_kernel,
        grid=(nh, nq),
        in_specs=[
            pl.BlockSpec((BQ, HP * dh), lambda hh, i: (i, hh)),
            pl.BlockSpec((s, HP * dh), lambda hh, i: (0, hh)),
            pl.BlockSpec((s, HP * dh), lambda hh, i: (0, hh)),
            pl.BlockSpec((HP, BQ, s), lambda hh, i: (hh, i, 0)),
        ],
        out_specs=pl.BlockSpec((BQ, 1, HP * dh), lambda hh, i: (i, 0, hh)),
        out_shape=jax.ShapeDtypeStruct((s, b, hd), jnp.float32),
        scratch_shapes=[
            pltpu.VMEM((s, HP * dh), jnp.bfloat16),
            pltpu.VMEM((s, HP * dh), jnp.bfloat16),
        ],
    )(qb, kb, vb, mask)

    return out


# speedup vs baseline: 1.2643x; 1.0179x over previous
import math

import jax
import jax.numpy as jnp
from jax.experimental import pallas as pl
from jax.experimental.pallas import tpu as pltpu

BQ = 1024
HP = 2


def _attn_block_kernel(q_ref, k_ref, v_ref, m_ref, o_ref, kb_ref, vb_ref):
    dh = q_ref.shape[-1] // HP
    c = math.log2(math.e) / math.sqrt(dh)

    @pl.when(pl.program_id(1) == 0)
    def _():
        kb_ref[...] = k_ref[...].astype(jnp.bfloat16)
        vb_ref[...] = v_ref[...].astype(jnp.bfloat16)

    qp = (q_ref[...] * c).astype(jnp.bfloat16)
    kp = kb_ref[...]
    vp = vb_ref[...]
    outs = []
    for j in range(HP):
        qj = qp[:, j * dh:(j + 1) * dh]
        kj = kp[:, j * dh:(j + 1) * dh]
        vj = vp[:, j * dh:(j + 1) * dh]
        mj = m_ref[j]
        s2 = jax.lax.dot_general(
            qj, kj, (((1,), (1,)), ((), ())), preferred_element_type=jnp.float32
        )
        mx = jnp.max(s2, axis=-1, keepdims=True)
        p = jnp.exp2(s2 - mx) * mj
        d = jnp.sum(p, axis=-1, keepdims=True)
        o = jax.lax.dot_general(
            p.astype(jnp.bfloat16), vj, (((1,), (0,)), ((), ())),
            preferred_element_type=jnp.float32,
        )
        outs.append(o / d)
    o_ref[:, 0, :] = jnp.concatenate(outs, axis=-1)


def kernel(query, key, value, mask):
    b, s, h, dh = query.shape
    hd = h * dh
    nq = s // BQ
    nh = h // HP

    qb = query.reshape(s, hd)
    kb = key.reshape(s, hd)
    vb = value.reshape(s, hd)

    out = pl.pallas_call(
        _attn_block_kernel,
        grid=(nh, nq),
        in_specs=[
            pl.BlockSpec((BQ, HP * dh), lambda hh, i: (i, hh)),
            pl.BlockSpec((s, HP * dh), lambda hh, i: (0, hh)),
            pl.BlockSpec((s, HP * dh), lambda hh, i: (0, hh)),
            pl.BlockSpec((HP, BQ, s), lambda hh, i: (hh, i, 0)),
        ],
        out_specs=pl.BlockSpec((BQ, 1, HP * dh), lambda hh, i: (i, 0, hh)),
        out_shape=jax.ShapeDtypeStruct((s, b, hd), jnp.float32),
        scratch_shapes=[
            pltpu.VMEM((s, HP * dh), jnp.bfloat16),
            pltpu.VMEM((s, HP * dh), jnp.bfloat16),
        ],
    )(qb, kb, vb, mask)

    return out


# speedup vs baseline: 1.4529x; 1.1492x over previous
import math

import jax
import jax.numpy as jnp
from jax.experimental import pallas as pl
from jax.experimental.pallas import tpu as pltpu

BQ = 1024
HP = 2


def _attn_block_kernel(q_ref, k_ref, v_ref, m_ref, o_ref, kb_ref, vb_ref,
                       kn_ref):
    hpdh = q_ref.shape[-1]
    dh = hpdh // HP
    c = math.log2(math.e) / math.sqrt(dh)
    s_len = k_ref.shape[0]

    @pl.when(pl.program_id(1) == 0)
    def _init():
        k = k_ref[...]
        kb_ref[...] = k.astype(jnp.bfloat16)
        v = v_ref[...]
        ecol = (jax.lax.broadcasted_iota(jnp.int32, (s_len, dh), 1) == 0
                ).astype(jnp.bfloat16)
        for j in range(HP):
            vj = v[:, j * dh:(j + 1) * dh]
            vb_ref[:, 2 * j * dh:(2 * j + 1) * dh] = vj.astype(jnp.bfloat16)
            vb_ref[:, (2 * j + 1) * dh:(2 * j + 2) * dh] = ecol
            kj = k[:, j * dh:(j + 1) * dh] * c
            kn2 = jnp.max(jnp.sum(kj * kj, axis=-1))
            kn_ref[j] = jnp.sqrt(kn2)

    qp = (q_ref[...] * c).astype(jnp.bfloat16)
    outs = []
    for j in range(HP):
        qj32 = q_ref[:, j * dh:(j + 1) * dh] * c
        qn = jnp.sqrt(jnp.sum(qj32 * qj32, axis=-1, keepdims=True))
        bound = qn * kn_ref[j]
        qj = qp[:, j * dh:(j + 1) * dh]
        kj = kb_ref[:, j * dh:(j + 1) * dh]
        s2 = jax.lax.dot_general(
            qj, kj, (((1,), (1,)), ((), ())), preferred_element_type=jnp.float32
        )
        p = jnp.exp2(s2 - bound) * m_ref[j]
        oe = jax.lax.dot_general(
            p.astype(jnp.bfloat16), vb_ref[:, 2 * j * dh:(2 * j + 2) * dh],
            (((1,), (0,)), ((), ())), preferred_element_type=jnp.float32,
        )
        outs.append(oe[:, 0:dh] / oe[:, dh:dh + 1])
    o_ref[:, 0, :] = jnp.concatenate(outs, axis=-1)


def kernel(query, key, value, mask):
    b, s, h, dh = query.shape
    hd = h * dh
    nq = s // BQ
    nh = h // HP

    qb = query.reshape(s, hd)
    kb = key.reshape(s, hd)
    vb = value.reshape(s, hd)

    out = pl.pallas_call(
        _attn_block_kernel,
        grid=(nh, nq),
        in_specs=[
            pl.BlockSpec((BQ, HP * dh), lambda hh, i: (i, hh)),
            pl.BlockSpec((s, HP * dh), lambda hh, i: (0, hh)),
            pl.BlockSpec((s, HP * dh), lambda hh, i: (0, hh)),
            pl.BlockSpec((HP, BQ, s), lambda hh, i: (hh, i, 0)),
        ],
        out_specs=pl.BlockSpec((BQ, 1, HP * dh), lambda hh, i: (i, 0, hh)),
        out_shape=jax.ShapeDtypeStruct((s, b, hd), jnp.float32),
        scratch_shapes=[
            pltpu.VMEM((s, HP * dh), jnp.bfloat16),
            pltpu.VMEM((s, HP * 2 * dh), jnp.bfloat16),
            pltpu.SMEM((HP,), jnp.float32),
        ],
    )(qb, kb, vb, mask)

    return out


# speedup vs baseline: 1.4540x; 1.0007x over previous
import math

import jax
import jax.numpy as jnp
from jax.experimental import pallas as pl
from jax.experimental.pallas import tpu as pltpu

BQ = 512
HP = 4


def _attn_block_kernel(q_ref, k_ref, v_ref, m_ref, o_ref, kb_ref, vb_ref,
                       kn_ref):
    hpdh = q_ref.shape[-1]
    dh = hpdh // HP
    c = math.log2(math.e) / math.sqrt(dh)
    s_len = k_ref.shape[0]

    @pl.when(pl.program_id(1) == 0)
    def _init():
        k = k_ref[...]
        kb_ref[...] = k.astype(jnp.bfloat16)
        v = v_ref[...]
        ecol = (jax.lax.broadcasted_iota(jnp.int32, (s_len, dh), 1) == 0
                ).astype(jnp.bfloat16)
        for j in range(HP):
            vj = v[:, j * dh:(j + 1) * dh]
            vb_ref[:, 2 * j * dh:(2 * j + 1) * dh] = vj.astype(jnp.bfloat16)
            vb_ref[:, (2 * j + 1) * dh:(2 * j + 2) * dh] = ecol
            kj = k[:, j * dh:(j + 1) * dh] * c
            kn2 = jnp.max(jnp.sum(kj * kj, axis=-1))
            kn_ref[j] = jnp.sqrt(kn2)

    qp = (q_ref[...] * c).astype(jnp.bfloat16)
    outs = []
    for j in range(HP):
        qj32 = q_ref[:, j * dh:(j + 1) * dh] * c
        qn = jnp.sqrt(jnp.sum(qj32 * qj32, axis=-1, keepdims=True))
        bound = qn * kn_ref[j]
        qj = qp[:, j * dh:(j + 1) * dh]
        kj = kb_ref[:, j * dh:(j + 1) * dh]
        s2 = jax.lax.dot_general(
            qj, kj, (((1,), (1,)), ((), ())), preferred_element_type=jnp.float32
        )
        p = jnp.exp2(s2 - bound) * m_ref[j]
        oe = jax.lax.dot_general(
            p.astype(jnp.bfloat16), vb_ref[:, 2 * j * dh:(2 * j + 2) * dh],
            (((1,), (0,)), ((), ())), preferred_element_type=jnp.float32,
        )
        outs.append(oe[:, 0:dh] / oe[:, dh:dh + 1])
    o_ref[:, 0, :] = jnp.concatenate(outs, axis=-1)


def kernel(query, key, value, mask):
    b, s, h, dh = query.shape
    hd = h * dh
    nq = s // BQ
    nh = h // HP

    qb = query.reshape(s, hd)
    kb = key.reshape(s, hd)
    vb = value.reshape(s, hd)

    out = pl.pallas_call(
        _attn_block_kernel,
        grid=(nh, nq),
        in_specs=[
            pl.BlockSpec((BQ, HP * dh), lambda hh, i: (i, hh)),
            pl.BlockSpec((s, HP * dh), lambda hh, i: (0, hh)),
            pl.BlockSpec((s, HP * dh), lambda hh, i: (0, hh)),
            pl.BlockSpec((HP, BQ, s), lambda hh, i: (hh, i, 0)),
        ],
        out_specs=pl.BlockSpec((BQ, 1, HP * dh), lambda hh, i: (i, 0, hh)),
        out_shape=jax.ShapeDtypeStruct((s, b, hd), jnp.float32),
        scratch_shapes=[
            pltpu.VMEM((s, HP * dh), jnp.bfloat16),
            pltpu.VMEM((s, HP * 2 * dh), jnp.bfloat16),
            pltpu.SMEM((HP,), jnp.float32),
        ],
    )(qb, kb, vb, mask)

    return out


# speedup vs baseline: 1.5103x; 1.0388x over previous
import math

import jax
import jax.numpy as jnp
from jax.experimental import pallas as pl
from jax.experimental.pallas import tpu as pltpu

BQ = 512
HP = 4


def _attn_block_kernel(q_ref, k_ref, v_ref, m_ref, o_ref, vb_ref, kn_ref):
    hpdh = q_ref.shape[-1]
    dh = hpdh // HP
    s_len = k_ref.shape[0]

    @pl.when(pl.program_id(1) == 0)
    def _init():
        v = v_ref[...]
        ecol = (jax.lax.broadcasted_iota(jnp.int32, (s_len, dh), 1) == 0
                ).astype(jnp.bfloat16)
        k = k_ref[...].astype(jnp.float32)
        for j in range(HP):
            vb_ref[:, 2 * j * dh:(2 * j + 1) * dh] = v[:, j * dh:(j + 1) * dh]
            vb_ref[:, (2 * j + 1) * dh:(2 * j + 2) * dh] = ecol
            kj = k[:, j * dh:(j + 1) * dh]
            kn_ref[j] = jnp.sqrt(jnp.max(jnp.sum(kj * kj, axis=-1)))

    qp = q_ref[...]
    q32 = qp.astype(jnp.float32)
    outs = []
    for j in range(HP):
        qj32 = q32[:, j * dh:(j + 1) * dh]
        qn = jnp.sqrt(jnp.sum(qj32 * qj32, axis=-1, keepdims=True))
        bound = qn * kn_ref[j]
        qj = qp[:, j * dh:(j + 1) * dh]
        kj = k_ref[:, j * dh:(j + 1) * dh]
        s2 = jax.lax.dot_general(
            qj, kj, (((1,), (1,)), ((), ())), preferred_element_type=jnp.float32
        )
        p = jnp.exp2(s2 - bound) * m_ref[j]
        oe = jax.lax.dot_general(
            p.astype(jnp.bfloat16), vb_ref[:, 2 * j * dh:(2 * j + 2) * dh],
            (((1,), (0,)), ((), ())), preferred_element_type=jnp.float32,
        )
        outs.append(oe[:, 0:dh] / oe[:, dh:dh + 1])
    o_ref[:, 0, :] = jnp.concatenate(outs, axis=-1)


def kernel(query, key, value, mask):
    b, s, h, dh = query.shape
    hd = h * dh
    nq = s // BQ
    nh = h // HP
    c = math.log2(math.e) / math.sqrt(dh)

    qb = (query.reshape(s, hd) * c).astype(jnp.bfloat16)
    kb = key.reshape(s, hd).astype(jnp.bfloat16)
    vb = value.reshape(s, hd).astype(jnp.bfloat16)

    out = pl.pallas_call(
        _attn_block_kernel,
        grid=(nh, nq),
        in_specs=[
            pl.BlockSpec((BQ, HP * dh), lambda hh, i: (i, hh)),
            pl.BlockSpec((s, HP * dh), lambda hh, i: (0, hh)),
            pl.BlockSpec((s, HP * dh), lambda hh, i: (0, hh)),
            pl.BlockSpec((HP, BQ, s), lambda hh, i: (hh, i, 0)),
        ],
        out_specs=pl.BlockSpec((BQ, 1, HP * dh), lambda hh, i: (i, 0, hh)),
        out_shape=jax.ShapeDtypeStruct((s, b, hd), jnp.float32),
        scratch_shapes=[
            pltpu.VMEM((s, HP * 2 * dh), jnp.bfloat16),
            pltpu.SMEM((HP,), jnp.float32),
        ],
    )(qb, kb, vb, mask)

    return out
